# baseline (speedup 1.0000x reference)
import jax
import jax.numpy as jnp
import numpy as np
from jax.experimental import pallas as pl
from jax.experimental.pallas import tpu as pltpu

_B, _CIN, _COUT, _H, _W = 8, 8, 16, 256, 256
_TH = 128

_mesh = pltpu.create_tensorcore_mesh("core", num_cores=2)


def _body(c_ref, s_ref, x_ref, o_ref):
    for o in range(_COUT):
        acc = None
        for j in range(_CIN):
            d = x_ref[0, j] - c_ref[o, j]
            e = jnp.exp2(d * d * s_ref[o, j])
            acc = e if acc is None else acc + e
        o_ref[0, o] = acc


def kernel(x, bin_centers, bin_widths):
    s = (-np.log2(np.e) * 0.5) / (bin_widths * bin_widths)

    def run(refs):
        c_hbm, s_hbm, x_hbm, o_hbm = refs

        @pl.core_map(_mesh)
        def _():
            pltpu.emit_pipeline(
                _body,
                grid=(_B, _H // _TH),
                in_specs=[
                    pl.BlockSpec((_COUT, _CIN), lambda b, h: (0, 0),
                                 memory_space=pltpu.SMEM),
                    pl.BlockSpec((_COUT, _CIN), lambda b, h: (0, 0),
                                 memory_space=pltpu.SMEM),
                    pl.BlockSpec((1, _CIN, _TH, _W), lambda b, h: (b, 0, h, 0)),
                ],
                out_specs=[
                    pl.BlockSpec((1, _COUT, _TH, _W), lambda b, h: (b, 0, h, 0)),
                ],
                core_axis_name="core",
                dimension_semantics=(pltpu.PARALLEL, pltpu.ARBITRARY),
            )(c_hbm, s_hbm, x_hbm, o_hbm)

    out_init = jax.lax.empty((_B, _COUT, _H, _W), jnp.float32)
    _, _, _, out = pl.run_state(run)((bin_centers, s, x, out_init))
    return out


# speedup vs baseline: 1.0640x; 1.0640x over previous
import jax
import jax.numpy as jnp
import numpy as np
from jax.experimental import pallas as pl
from jax.experimental.pallas import tpu as pltpu

_B, _CIN, _COUT, _H, _W = 8, 8, 16, 256, 256
_TH = 64


def _hist_kernel(c_ref, s_ref, x_ref, o_ref):
    for o in range(_COUT):
        acc = None
        for j in range(_CIN):
            d = x_ref[0, j] - c_ref[o, j]
            e = jnp.exp2(d * d * s_ref[o, j])
            acc = e if acc is None else acc + e
        o_ref[0, o] = acc


def kernel(x, bin_centers, bin_widths):
    s = (-np.log2(np.e) * 0.5) / (bin_widths * bin_widths)
    grid = (_B, _H // _TH)
    return pl.pallas_call(
        _hist_kernel,
        out_shape=jax.ShapeDtypeStruct((_B, _COUT, _H, _W), jnp.float32),
        grid=grid,
        in_specs=[
            pl.BlockSpec(memory_space=pltpu.SMEM),
            pl.BlockSpec(memory_space=pltpu.SMEM),
            pl.BlockSpec((1, _CIN, _TH, _W), lambda b, h: (b, 0, h, 0)),
        ],
        out_specs=pl.BlockSpec((1, _COUT, _TH, _W), lambda b, h: (b, 0, h, 0)),
        compiler_params=pltpu.CompilerParams(
            dimension_semantics=("arbitrary", "arbitrary"),
        ),
        name="rbf_soft_histogram",
    )(bin_centers, s, x)


# speedup vs baseline: 1.1151x; 1.0480x over previous
import jax
import jax.numpy as jnp
import numpy as np
from jax.experimental import pallas as pl
from jax.experimental.pallas import tpu as pltpu

_B, _CIN, _COUT, _H, _W = 8, 8, 16, 256, 256
_TH = 128


def _hist_kernel(c_ref, s_ref, x_ref, o_ref):
    for o in range(_COUT):
        acc = None
        for j in range(_CIN):
            d = x_ref[0, j] - c_ref[o, j]
            e = jnp.exp2(d * d * s_ref[o, j])
            acc = e if acc is None else acc + e
        o_ref[0, o] = acc


def kernel(x, bin_centers, bin_widths):
    s = (-np.log2(np.e) * 0.5) / (bin_widths * bin_widths)
    grid = (_B, _H // _TH)
    return pl.pallas_call(
        _hist_kernel,
        out_shape=jax.ShapeDtypeStruct((_B, _COUT, _H, _W), jnp.float32),
        grid=grid,
        in_specs=[
            pl.BlockSpec(memory_space=pltpu.SMEM),
            pl.BlockSpec(memory_space=pltpu.SMEM),
            pl.BlockSpec((1, _CIN, _TH, _W), lambda b, h: (b, 0, h, 0)),
        ],
        out_specs=pl.BlockSpec((1, _COUT, _TH, _W), lambda b, h: (b, 0, h, 0)),
        compiler_params=pltpu.CompilerParams(
            dimension_semantics=("arbitrary", "arbitrary"),
        ),
        name="rbf_soft_histogram",
    )(bin_centers, s, x)
